# baseline (speedup 1.0000x reference)
import functools

import jax
import jax.numpy as jnp
from jax import lax
from jax.experimental import pallas as pl
from jax.experimental.pallas import tpu as pltpu
from jax.experimental.pallas import tpu_sc as plsc

_TEMP = 0.07
_NHARD = 16
_ROWS_PER_BLOCK = 512
_EPS = 1e-12


def _tc_body(feat_ref, pe_ref, cbt_ref, out_ref):
    f = feat_ref[...]
    pe = pe_ref[...]
    cbt = cbt_ref[...]
    r = f.shape[0]

    b2 = jnp.sum(cbt * cbt, axis=0, keepdims=True)
    sc_b = (1.0 / _TEMP) / jnp.maximum(jnp.sqrt(b2), _EPS)
    qn = jnp.sqrt(jnp.sum(f * f, axis=1, keepdims=True))
    q = f / jnp.maximum(qn, _EPS)

    a = jnp.concatenate([-2.0 * pe, q], axis=0)
    pq = jnp.dot(a, cbt, preferred_element_type=jnp.float32)
    rank = b2 + pq[:r, :]
    s = pq[r:, :] * sc_b

    work = rank
    tmin = None
    tk = None
    for i in range(_NHARD + 1):
        m = jnp.min(work, axis=1, keepdims=True)
        if i == 0:
            tmin = m
        tk = m
        if i < _NHARD:
            work = jnp.where(work <= m, jnp.float32(jnp.inf), work)

    neg_mask = jnp.logical_and(rank <= tk, rank > tmin)
    w = jnp.where(neg_mask, jnp.exp(s), 0.0)
    negsum = jnp.sum(w, axis=1, keepdims=True)

    a2 = jnp.sum(pe * pe, axis=1, keepdims=True)
    kpos = pe / jnp.maximum(jnp.sqrt(a2), _EPS)
    pos_l = jnp.sum(q * kpos, axis=1, keepdims=True) * (1.0 / _TEMP)
    row_loss = jnp.log(jnp.exp(pos_l) + negsum) - pos_l
    out_ref[...] = jnp.full((1, 1, 128), jnp.sum(row_loss), jnp.float32)


def _tc_loss(feat2d, pe, cbt):
    bl, c = feat2d.shape
    k = cbt.shape[1]
    r = _ROWS_PER_BLOCK
    nb = bl // r
    partial = pl.pallas_call(
        _tc_body,
        grid=(nb,),
        in_specs=[
            pl.BlockSpec((r, c), lambda i: (i, 0)),
            pl.BlockSpec((r, c), lambda i: (i, 0)),
            pl.BlockSpec((c, k), lambda i: (0, 0)),
        ],
        out_specs=pl.BlockSpec((1, 1, 128), lambda i: (i, 0, 0)),
        out_shape=jax.ShapeDtypeStruct((nb, 1, 128), jnp.float32),
    )(feat2d, pe, cbt)
    return jnp.sum(partial[:, 0, 0]) / bl


def _sc_gather(table, idx_flat):
    info = plsc.get_sparse_core_info()
    nw = info.num_cores * info.num_subcores
    b = idx_flat.shape[0]
    d = table.shape[1]
    b_per_w = b // nw
    mesh = plsc.VectorSubcoreMesh(core_axis_name="c", subcore_axis_name="s")

    @functools.partial(
        pl.kernel,
        mesh=mesh,
        out_type=jax.ShapeDtypeStruct((b, d), jnp.float32),
        scratch_types=[
            pltpu.VMEM((b_per_w,), jnp.int32),
            pltpu.VMEM((b_per_w, d), jnp.float32),
            pltpu.SemaphoreType.DMA,
        ],
    )
    def k(table_hbm, idx_hbm, out_hbm, idx_v, rows_v, sem):
        wid = lax.axis_index("s") * info.num_cores + lax.axis_index("c")
        base = wid * b_per_w
        pltpu.sync_copy(idx_hbm.at[pl.ds(base, b_per_w)], idx_v)
        pltpu.async_copy(table_hbm.at[idx_v], rows_v, sem).wait()
        pltpu.sync_copy(rows_v, out_hbm.at[pl.ds(base, b_per_w)])

    return k(table, idx_flat)


def kernel(decoder_feat, codebook, positive_indices):
    b, l, c = decoder_feat.shape
    k = codebook.shape[0]
    idx = jnp.clip(positive_indices.reshape(-1), 0, k - 1).astype(jnp.int32)
    cb_pad = jnp.pad(codebook, ((0, 0), (0, 128 - c)))
    pos_emb = _sc_gather(cb_pad, idx)[:, :c]
    feat2d = decoder_feat.reshape(b * l, c)
    return _tc_loss(feat2d, pos_emb, codebook.T)


# speedup vs baseline: 2.2425x; 2.2425x over previous
import functools

import jax
import jax.numpy as jnp
from jax import lax
from jax.experimental import pallas as pl
from jax.experimental.pallas import tpu as pltpu
from jax.experimental.pallas import tpu_sc as plsc

_TEMP = 0.07
_NHARD = 16
_ROWS_PER_BLOCK = 256
_EPS = 1e-12
_SLAB = 128
_SUMK = 5


def _tree_min(xs):
    xs = list(xs)
    while len(xs) > 1:
        nxt = [jnp.minimum(xs[i], xs[i + 1]) for i in range(0, len(xs) - 1, 2)]
        if len(xs) % 2:
            nxt.append(xs[-1])
        xs = nxt
    return xs[0]


def _extract_thresholds(arr):
    wk = arr
    t0 = None
    t = None
    for i in range(_NHARD + 1):
        m = jnp.min(wk, axis=1, keepdims=True)
        if i == 0:
            t0 = m
        t = m
        if i < _NHARD:
            wk = jnp.where(wk <= m, jnp.float32(jnp.inf), wk)
    return t0, t


def _tc_body(feat_ref, pe_ref, cbt_ref, out_ref):
    f = feat_ref[...]
    pe = pe_ref[...]
    cbt = cbt_ref[...]
    r = f.shape[0]

    b2 = jnp.sum(cbt * cbt, axis=0, keepdims=True)
    sc_b = (1.0 / _TEMP) / jnp.maximum(jnp.sqrt(b2), _EPS)
    qn = jnp.sqrt(jnp.sum(f * f, axis=1, keepdims=True))
    q = f / jnp.maximum(qn, _EPS)

    a = jnp.concatenate([-2.0 * pe, q], axis=0)
    pq = jnp.dot(a, cbt, preferred_element_type=jnp.float32)
    rank = b2 + pq[:r, :]
    s = pq[r:, :] * sc_b

    nslab = rank.shape[1] // _SLAB
    slabs = [rank[:, i * _SLAB:(i + 1) * _SLAB] for i in range(nslab)]
    mins = [_tree_min(slabs)]
    for _ in range(_SUMK - 1):
        slabs = [jnp.where(sl == mins[-1], jnp.float32(jnp.inf), sl)
                 for sl in slabs]
        mins.append(_tree_min(slabs))
    summary = jnp.concatenate(mins, axis=1)

    tmin_f, tk_f = _extract_thresholds(summary)
    viol = jnp.any(mins[-1] <= tk_f)
    thr = lax.cond(
        viol,
        lambda: jnp.concatenate(_extract_thresholds(rank), axis=1),
        lambda: jnp.concatenate((tmin_f, tk_f), axis=1),
    )
    tmin = thr[:, 0:1]
    tk = thr[:, 1:2]

    neg_mask = jnp.logical_and(rank <= tk, rank > tmin)
    w = jnp.where(neg_mask, jnp.exp(s), 0.0)
    negsum = jnp.sum(w, axis=1, keepdims=True)

    a2 = jnp.sum(pe * pe, axis=1, keepdims=True)
    kpos = pe / jnp.maximum(jnp.sqrt(a2), _EPS)
    pos_l = jnp.sum(q * kpos, axis=1, keepdims=True) * (1.0 / _TEMP)
    row_loss = jnp.log(jnp.exp(pos_l) + negsum) - pos_l
    out_ref[...] = jnp.full((1, 1, 128), jnp.sum(row_loss), jnp.float32)


def _tc_loss(feat2d, pe, cbt):
    bl, c = feat2d.shape
    k = cbt.shape[1]
    r = _ROWS_PER_BLOCK
    nb = bl // r
    partial = pl.pallas_call(
        _tc_body,
        grid=(nb,),
        in_specs=[
            pl.BlockSpec((r, c), lambda i: (i, 0)),
            pl.BlockSpec((r, c), lambda i: (i, 0)),
            pl.BlockSpec((c, k), lambda i: (0, 0)),
        ],
        out_specs=pl.BlockSpec((1, 1, 128), lambda i: (i, 0, 0)),
        out_shape=jax.ShapeDtypeStruct((nb, 1, 128), jnp.float32),
    )(feat2d, pe, cbt)
    return jnp.sum(partial[:, 0, 0]) / bl


def _sc_gather(table, idx_flat):
    info = plsc.get_sparse_core_info()
    nw = info.num_cores * info.num_subcores
    b = idx_flat.shape[0]
    d = table.shape[1]
    b_per_w = b // nw
    mesh = plsc.VectorSubcoreMesh(core_axis_name="c", subcore_axis_name="s")

    @functools.partial(
        pl.kernel,
        mesh=mesh,
        out_type=jax.ShapeDtypeStruct((b, d), jnp.float32),
        scratch_types=[
            pltpu.VMEM((b_per_w,), jnp.int32),
            pltpu.VMEM((b_per_w, d), jnp.float32),
            pltpu.SemaphoreType.DMA,
        ],
    )
    def k(table_hbm, idx_hbm, out_hbm, idx_v, rows_v, sem):
        wid = lax.axis_index("s") * info.num_cores + lax.axis_index("c")
        base = wid * b_per_w
        pltpu.sync_copy(idx_hbm.at[pl.ds(base, b_per_w)], idx_v)
        pltpu.async_copy(table_hbm.at[idx_v], rows_v, sem).wait()
        pltpu.sync_copy(rows_v, out_hbm.at[pl.ds(base, b_per_w)])

    return k(table, idx_flat)


def kernel(decoder_feat, codebook, positive_indices):
    b, l, c = decoder_feat.shape
    k = codebook.shape[0]
    idx = jnp.clip(positive_indices.reshape(-1), 0, k - 1).astype(jnp.int32)
    cb_pad = jnp.pad(codebook, ((0, 0), (0, 128 - c)))
    pos_emb = _sc_gather(cb_pad, idx)[:, :c]
    feat2d = decoder_feat.reshape(b * l, c)
    return _tc_loss(feat2d, pos_emb, codebook.T)


# speedup vs baseline: 2.3107x; 1.0304x over previous
import functools

import jax
import jax.numpy as jnp
from jax import lax
from jax.experimental import pallas as pl
from jax.experimental.pallas import tpu as pltpu
from jax.experimental.pallas import tpu_sc as plsc

_TEMP = 0.07
_NHARD = 16
_ROWS_PER_BLOCK = 256
_EPS = 1e-12
_SLAB = 128
_SUMK = 5


def _tree_min(xs):
    xs = list(xs)
    while len(xs) > 1:
        nxt = [jnp.minimum(xs[i], xs[i + 1]) for i in range(0, len(xs) - 1, 2)]
        if len(xs) % 2:
            nxt.append(xs[-1])
        xs = nxt
    return xs[0]


def _extract_threshold(arr):
    wk = arr
    t = None
    for i in range(_NHARD + 1):
        t = jnp.min(wk, axis=1, keepdims=True)
        if i < _NHARD:
            wk = jnp.where(wk <= t, jnp.float32(jnp.inf), wk)
    return t


def _tc_body(feat_ref, pe_ref, cbt_ref, out_ref):
    f = feat_ref[...]
    pe = pe_ref[...]
    cbt = cbt_ref[...]
    r = f.shape[0]

    b2 = jnp.sum(cbt * cbt, axis=0, keepdims=True)
    sc_b = (1.0 / _TEMP) / jnp.maximum(jnp.sqrt(b2), _EPS)
    qn = jnp.sqrt(jnp.sum(f * f, axis=1, keepdims=True))
    q = f / jnp.maximum(qn, _EPS)

    a = jnp.concatenate([-2.0 * pe, q], axis=0)
    pq = jnp.dot(a, cbt, preferred_element_type=jnp.float32)
    rank = b2 + pq[:r, :]
    s = pq[r:, :] * sc_b

    nslab = rank.shape[1] // _SLAB
    slabs = [rank[:, i * _SLAB:(i + 1) * _SLAB] for i in range(nslab)]
    mins = [_tree_min(slabs)]
    for _ in range(_SUMK - 1):
        slabs = [jnp.where(sl == mins[-1], jnp.float32(jnp.inf), sl)
                 for sl in slabs]
        mins.append(_tree_min(slabs))
    summary = jnp.concatenate(mins, axis=1)

    tk_f = _extract_threshold(summary)
    viol = jnp.any(mins[-1] <= tk_f)
    tk = lax.cond(viol, lambda: _extract_threshold(rank), lambda: tk_f)

    w = jnp.where(rank <= tk, jnp.exp(s), 0.0)
    den = jnp.sum(w, axis=1, keepdims=True)

    a2 = jnp.sum(pe * pe, axis=1, keepdims=True)
    kpos = pe / jnp.maximum(jnp.sqrt(a2), _EPS)
    pos_l = jnp.sum(q * kpos, axis=1, keepdims=True) * (1.0 / _TEMP)
    row_loss = jnp.log(den) - pos_l
    out_ref[...] = jnp.full((1, 1, 128), jnp.sum(row_loss), jnp.float32)


def _tc_loss(feat2d, pe, cbt):
    bl, c = feat2d.shape
    k = cbt.shape[1]
    r = _ROWS_PER_BLOCK
    nb = bl // r
    partial = pl.pallas_call(
        _tc_body,
        grid=(nb,),
        in_specs=[
            pl.BlockSpec((r, c), lambda i: (i, 0)),
            pl.BlockSpec((r, c), lambda i: (i, 0)),
            pl.BlockSpec((c, k), lambda i: (0, 0)),
        ],
        out_specs=pl.BlockSpec((1, 1, 128), lambda i: (i, 0, 0)),
        out_shape=jax.ShapeDtypeStruct((nb, 1, 128), jnp.float32),
    )(feat2d, pe, cbt)
    return jnp.sum(partial[:, 0, 0]) / bl


def _sc_gather(table, idx_flat):
    info = plsc.get_sparse_core_info()
    nw = info.num_cores * info.num_subcores
    b = idx_flat.shape[0]
    d = table.shape[1]
    b_per_w = b // nw
    mesh = plsc.VectorSubcoreMesh(core_axis_name="c", subcore_axis_name="s")

    @functools.partial(
        pl.kernel,
        mesh=mesh,
        out_type=jax.ShapeDtypeStruct((b, d), jnp.float32),
        scratch_types=[
            pltpu.VMEM((b_per_w,), jnp.int32),
            pltpu.VMEM((b_per_w, d), jnp.float32),
            pltpu.SemaphoreType.DMA,
        ],
    )
    def k(table_hbm, idx_hbm, out_hbm, idx_v, rows_v, sem):
        wid = lax.axis_index("s") * info.num_cores + lax.axis_index("c")
        base = wid * b_per_w
        pltpu.sync_copy(idx_hbm.at[pl.ds(base, b_per_w)], idx_v)
        pltpu.async_copy(table_hbm.at[idx_v], rows_v, sem).wait()
        pltpu.sync_copy(rows_v, out_hbm.at[pl.ds(base, b_per_w)])

    return k(table, idx_flat)


def kernel(decoder_feat, codebook, positive_indices):
    b, l, c = decoder_feat.shape
    k = codebook.shape[0]
    idx = jnp.clip(positive_indices.reshape(-1), 0, k - 1).astype(jnp.int32)
    cb_pad = jnp.pad(codebook, ((0, 0), (0, 128 - c)))
    pos_emb = _sc_gather(cb_pad, idx)[:, :c]
    feat2d = decoder_feat.reshape(b * l, c)
    return _tc_loss(feat2d, pos_emb, codebook.T)


# speedup vs baseline: 2.7082x; 1.1720x over previous
import functools

import jax
import jax.numpy as jnp
from jax import lax
from jax.experimental import pallas as pl
from jax.experimental.pallas import tpu as pltpu
from jax.experimental.pallas import tpu_sc as plsc

_TEMP = 0.07
_NHARD = 16
_ROWS_PER_BLOCK = 256
_EPS = 1e-12
_SLAB = 128
_SUMK = 5


def _tree_min(xs):
    xs = list(xs)
    while len(xs) > 1:
        nxt = [jnp.minimum(xs[i], xs[i + 1]) for i in range(0, len(xs) - 1, 2)]
        if len(xs) % 2:
            nxt.append(xs[-1])
        xs = nxt
    return xs[0]


def _ce(a, b):
    return jnp.minimum(a, b), jnp.maximum(a, b)


def _merge22(a, b):
    l1, h1 = _ce(a[0], b[0])
    l2, h2 = _ce(a[1], b[1])
    m1, m2 = _ce(h1, l2)
    return (l1, m1, m2, h2)


def _merge44_keep4(a, b):
    l1 = jnp.minimum(a[0], b[3])
    l2 = jnp.minimum(a[1], b[2])
    l3 = jnp.minimum(a[2], b[1])
    l4 = jnp.minimum(a[3], b[0])
    x1, x3 = _ce(l1, l3)
    x2, x4 = _ce(l2, l4)
    y1, y2 = _ce(x1, x2)
    y3, y4 = _ce(x3, x4)
    return (y1, y2, y3, y4)


def _top5_summary(slabs):
    pairs = [_ce(slabs[i], slabs[i + 1]) for i in range(0, len(slabs), 2)]
    quads = [_merge22(pairs[i], pairs[i + 1]) for i in range(0, len(pairs), 2)]
    while len(quads) > 1:
        quads = [_merge44_keep4(quads[i], quads[i + 1])
                 for i in range(0, len(quads), 2)]
    m1, m2, m3, m4 = quads[0]
    masked = [jnp.where(sl <= m4, jnp.float32(jnp.inf), sl) for sl in slabs]
    m5 = _tree_min(masked)
    return [m1, m2, m3, m4, m5]


def _extract_threshold(arr):
    wk = arr
    t = None
    for i in range(_NHARD + 1):
        t = jnp.min(wk, axis=1, keepdims=True)
        if i < _NHARD:
            wk = jnp.where(wk <= t, jnp.float32(jnp.inf), wk)
    return t


def _tc_body(feat_ref, pe_ref, cbt_ref, out_ref):
    f = feat_ref[...]
    pe = pe_ref[...]
    cbt = cbt_ref[...]
    r = f.shape[0]

    b2 = jnp.sum(cbt * cbt, axis=0, keepdims=True)
    sc_b = (1.0 / _TEMP) / jnp.maximum(jnp.sqrt(b2), _EPS)
    qn = jnp.sqrt(jnp.sum(f * f, axis=1, keepdims=True))
    q = f / jnp.maximum(qn, _EPS)

    a = jnp.concatenate([-2.0 * pe, q], axis=0)
    pq = jnp.dot(a, cbt, preferred_element_type=jnp.float32)
    rank = b2 + pq[:r, :]
    s = pq[r:, :] * sc_b

    nslab = rank.shape[1] // _SLAB
    slabs = [rank[:, i * _SLAB:(i + 1) * _SLAB] for i in range(nslab)]
    mins = _top5_summary(slabs)
    summary = jnp.concatenate(mins, axis=1)

    tk_f = _extract_threshold(summary)
    viol = jnp.any(mins[-1] <= tk_f)
    tk = lax.cond(viol, lambda: _extract_threshold(rank), lambda: tk_f)

    w = jnp.where(rank <= tk, jnp.exp(s), 0.0)
    den = jnp.sum(w, axis=1, keepdims=True)

    a2 = jnp.sum(pe * pe, axis=1, keepdims=True)
    kpos = pe / jnp.maximum(jnp.sqrt(a2), _EPS)
    pos_l = jnp.sum(q * kpos, axis=1, keepdims=True) * (1.0 / _TEMP)
    row_loss = jnp.log(den) - pos_l
    out_ref[...] = jnp.full((1, 1, 128), jnp.sum(row_loss), jnp.float32)


def _tc_loss(feat2d, pe, cbt):
    bl, c = feat2d.shape
    k = cbt.shape[1]
    r = _ROWS_PER_BLOCK
    nb = bl // r
    partial = pl.pallas_call(
        _tc_body,
        grid=(nb,),
        in_specs=[
            pl.BlockSpec((r, c), lambda i: (i, 0)),
            pl.BlockSpec((r, c), lambda i: (i, 0)),
            pl.BlockSpec((c, k), lambda i: (0, 0)),
        ],
        out_specs=pl.BlockSpec((1, 1, 128), lambda i: (i, 0, 0)),
        out_shape=jax.ShapeDtypeStruct((nb, 1, 128), jnp.float32),
    )(feat2d, pe, cbt)
    return jnp.sum(partial[:, 0, 0]) / bl


def _sc_gather(table, idx_flat):
    info = plsc.get_sparse_core_info()
    nw = info.num_cores * info.num_subcores
    b = idx_flat.shape[0]
    d = table.shape[1]
    b_per_w = b // nw
    mesh = plsc.VectorSubcoreMesh(core_axis_name="c", subcore_axis_name="s")

    @functools.partial(
        pl.kernel,
        mesh=mesh,
        out_type=jax.ShapeDtypeStruct((b, d), jnp.float32),
        scratch_types=[
            pltpu.VMEM((b_per_w,), jnp.int32),
            pltpu.VMEM((b_per_w, d), jnp.float32),
            pltpu.SemaphoreType.DMA,
        ],
    )
    def k(table_hbm, idx_hbm, out_hbm, idx_v, rows_v, sem):
        wid = lax.axis_index("s") * info.num_cores + lax.axis_index("c")
        base = wid * b_per_w
        pltpu.sync_copy(idx_hbm.at[pl.ds(base, b_per_w)], idx_v)
        pltpu.async_copy(table_hbm.at[idx_v], rows_v, sem).wait()
        pltpu.sync_copy(rows_v, out_hbm.at[pl.ds(base, b_per_w)])

    return k(table, idx_flat)


def kernel(decoder_feat, codebook, positive_indices):
    b, l, c = decoder_feat.shape
    k = codebook.shape[0]
    idx = jnp.clip(positive_indices.reshape(-1), 0, k - 1).astype(jnp.int32)
    cb_pad = jnp.pad(codebook, ((0, 0), (0, 128 - c)))
    pos_emb = _sc_gather(cb_pad, idx)[:, :c]
    feat2d = decoder_feat.reshape(b * l, c)
    return _tc_loss(feat2d, pos_emb, codebook.T)
